# baseline (speedup 1.0000x reference)
import functools

import jax
import jax.numpy as jnp
from jax.experimental import pallas as pl
from jax.experimental.pallas import tpu as pltpu

N = 100000
D = 128
G = 256
BLK = 10000


def _fused_body(x_ref, b_ref, W1, b1, W2, b2, Wm1, bm1, Wm2, bm2,
                out_ref, acc_ref):
    i = pl.program_id(0)
    x = x_ref[...].astype(jnp.bfloat16)
    hidden = jnp.maximum(
        jax.lax.dot(x, W1[...],
                    preferred_element_type=jnp.float32).astype(jnp.bfloat16)
        + b1[...],
        jnp.bfloat16(0.0))
    wg = (jax.lax.dot(hidden, W2[...],
                      preferred_element_type=jnp.float32).astype(jnp.bfloat16)
          + b2[...])
    w = wg[:, :D]
    g = wg[:, D:]
    h = jax.nn.sigmoid(g) * w

    ids = b_ref[0, 0, :]
    onehot = (jax.lax.broadcasted_iota(jnp.int32, (G, BLK), 0)
              == ids[None, :]).astype(jnp.bfloat16)
    part = jax.lax.dot(onehot, h, preferred_element_type=jnp.float32)

    @pl.when(i == 0)
    def _init():
        acc_ref[...] = part

    @pl.when(i > 0)
    def _accum():
        acc_ref[...] += part

    @pl.when(i == pl.num_programs(0) - 1)
    def _final():
        acc = acc_ref[...]
        m = jnp.maximum(jax.lax.dot(acc, Wm1[...], preferred_element_type=jnp.float32)
                        + bm1[...], 0.0)
        out_ref[...] = (jax.lax.dot(m, Wm2[...], preferred_element_type=jnp.float32)
                        + bm2[...])


@functools.partial(jax.jit, static_argnums=(2,))
def _run(x, batch_i32, nblk, Ww1, bw1, Ww2, bw2, Wg1, bg1, Wg2, bg2,
         Wm1, bm1, Wm2, bm2):
    b3 = batch_i32.reshape(nblk, 1, BLK)
    W1 = jnp.concatenate([Ww1, Wg1], axis=1).astype(jnp.bfloat16)
    b1 = jnp.concatenate([bw1, bg1]).reshape(1, 2 * D).astype(jnp.bfloat16)
    zero = jnp.zeros((D, D), jnp.float32)
    W2 = jnp.block([[Ww2, zero], [zero, Wg2]]).astype(jnp.bfloat16)
    b2 = jnp.concatenate([bw2, bg2]).reshape(1, 2 * D).astype(jnp.bfloat16)

    row_spec = pl.BlockSpec((BLK, D), lambda i: (i, 0))
    id_spec = pl.BlockSpec((1, 1, BLK), lambda i: (i, 0, 0))
    full = lambda *shape: pl.BlockSpec(shape, lambda i: (0,) * len(shape))
    return pl.pallas_call(
        _fused_body,
        grid=(nblk,),
        in_specs=[row_spec, id_spec,
                  full(D, 2 * D), full(1, 2 * D), full(2 * D, 2 * D),
                  full(1, 2 * D), full(D, D), full(1, D), full(D, D),
                  full(1, D)],
        out_specs=full(G, D),
        out_shape=jax.ShapeDtypeStruct((G, D), jnp.float32),
        scratch_shapes=[pltpu.VMEM((G, D), jnp.float32)],
    )(x, b3, W1, b1, W2, b2,
      Wm1, bm1.reshape(1, D), Wm2, bm2.reshape(1, D))


def kernel(x, batch, dim, Ww1, bw1, Ww2, bw2, Wg1, bg1, Wg2, bg2,
           Wm1, bm1, Wm2, bm2):
    del dim
    batch_i32 = batch.astype(jnp.int32)
    assert x.shape == (N, D) and N % BLK == 0
    return _run(x, batch_i32, N // BLK, Ww1, bw1, Ww2, bw2,
                Wg1, bg1, Wg2, bg2, Wm1, bm1, Wm2, bm2)


# speedup vs baseline: 1.0792x; 1.0792x over previous
import functools

import jax
import jax.numpy as jnp
from jax.experimental import pallas as pl
from jax.experimental.pallas import tpu as pltpu

N = 100000
D = 128
G = 256
BLK = 10000


LOCAL = 64


def _fused_body(meta_ref, x_ref, b_ref, W1, b1, W2, b2, Wm1, bm1, Wm2, bm2,
                out_ref, acc_ref):
    i = pl.program_id(0)
    x = x_ref[...].astype(jnp.bfloat16)
    hidden = jnp.maximum(
        jax.lax.dot(x, W1[...],
                    preferred_element_type=jnp.float32).astype(jnp.bfloat16)
        + b1[...],
        jnp.bfloat16(0.0))
    wg = (jax.lax.dot(hidden, W2[...],
                      preferred_element_type=jnp.float32).astype(jnp.bfloat16)
          + b2[...])
    w = wg[:, :D]
    g = wg[:, D:]
    h = jax.nn.sigmoid(g) * w

    ids = b_ref[0, 0, :]
    base = meta_ref[0, i]
    span = meta_ref[1, i]

    @pl.when(i == 0)
    def _init():
        acc_ref[...] = jnp.zeros_like(acc_ref)

    @pl.when(span < LOCAL)
    def _local():
        onehot = (jax.lax.broadcasted_iota(jnp.int32, (LOCAL, BLK), 0)
                  == (ids - base)[None, :]).astype(jnp.bfloat16)
        part = jax.lax.dot(onehot, h, preferred_element_type=jnp.float32)
        acc_ref[pl.ds(base, LOCAL), :] += part

    @pl.when(span >= LOCAL)
    def _full():
        onehot = (jax.lax.broadcasted_iota(jnp.int32, (G, BLK), 0)
                  == ids[None, :]).astype(jnp.bfloat16)
        part = jax.lax.dot(onehot, h, preferred_element_type=jnp.float32)
        acc_ref[pl.ds(0, G), :] += part

    @pl.when(i == pl.num_programs(0) - 1)
    def _final():
        acc = acc_ref[pl.ds(0, G), :]
        m = jnp.maximum(jax.lax.dot(acc, Wm1[...], preferred_element_type=jnp.float32)
                        + bm1[...], 0.0)
        out_ref[...] = (jax.lax.dot(m, Wm2[...], preferred_element_type=jnp.float32)
                        + bm2[...])


@functools.partial(jax.jit, static_argnums=(2,))
def _run(x, batch_i32, nblk, Ww1, bw1, Ww2, bw2, Wg1, bg1, Wg2, bg2,
         Wm1, bm1, Wm2, bm2):
    b3 = batch_i32.reshape(nblk, 1, BLK)
    W1 = jnp.concatenate([Ww1, Wg1], axis=1).astype(jnp.bfloat16)
    b1 = jnp.concatenate([bw1, bg1]).reshape(1, 2 * D).astype(jnp.bfloat16)
    zero = jnp.zeros((D, D), jnp.float32)
    W2 = jnp.block([[Ww2, zero], [zero, Wg2]]).astype(jnp.bfloat16)
    b2 = jnp.concatenate([bw2, bg2]).reshape(1, 2 * D).astype(jnp.bfloat16)

    first = batch_i32[::BLK]
    last = batch_i32[BLK - 1::BLK]
    base = jnp.bitwise_and(first, -8)
    meta = jnp.stack([base, last - base])

    row_spec = pl.BlockSpec((BLK, D), lambda i, m: (i, 0))
    id_spec = pl.BlockSpec((1, 1, BLK), lambda i, m: (i, 0, 0))
    full = lambda *shape: pl.BlockSpec(shape, lambda i, m: (0,) * len(shape))
    grid_spec = pltpu.PrefetchScalarGridSpec(
        num_scalar_prefetch=1,
        grid=(nblk,),
        in_specs=[row_spec, id_spec,
                  full(D, 2 * D), full(1, 2 * D), full(2 * D, 2 * D),
                  full(1, 2 * D), full(D, D), full(1, D), full(D, D),
                  full(1, D)],
        out_specs=full(G, D),
        scratch_shapes=[pltpu.VMEM((G + LOCAL, D), jnp.float32)],
    )
    return pl.pallas_call(
        _fused_body,
        grid_spec=grid_spec,
        out_shape=jax.ShapeDtypeStruct((G, D), jnp.float32),
    )(meta, x, b3, W1, b1, W2, b2,
      Wm1, bm1.reshape(1, D), Wm2, bm2.reshape(1, D))


def kernel(x, batch, dim, Ww1, bw1, Ww2, bw2, Wg1, bg1, Wg2, bg2,
           Wm1, bm1, Wm2, bm2):
    del dim
    batch_i32 = batch.astype(jnp.int32)
    assert x.shape == (N, D) and N % BLK == 0
    return _run(x, batch_i32, N // BLK, Ww1, bw1, Ww2, bw2,
                Wg1, bg1, Wg2, bg2, Wm1, bm1, Wm2, bm2)


# speedup vs baseline: 1.0814x; 1.0021x over previous
import functools

import jax
import jax.numpy as jnp
from jax.experimental import pallas as pl
from jax.experimental.pallas import tpu as pltpu

N = 100000
D = 128
G = 256
BLK = 10000


LOCAL = 64


def _fused_body(meta_ref, x_ref, b_ref, W1, b1, W2, b2, Wm1, bm1, Wm2, bm2,
                out_ref, acc_ref):
    i = pl.program_id(0)
    x = x_ref[...].astype(jnp.bfloat16)
    hidden = jnp.maximum(
        jax.lax.dot(x, W1[...],
                    preferred_element_type=jnp.float32).astype(jnp.bfloat16)
        + b1[...],
        jnp.bfloat16(0.0))
    wg = (jax.lax.dot(hidden, W2[...],
                      preferred_element_type=jnp.float32).astype(jnp.bfloat16)
          + b2[...])
    w = wg[:, :D]
    g = wg[:, D:]
    h = (jnp.bfloat16(0.5) * jnp.tanh(g * jnp.bfloat16(0.5))
         + jnp.bfloat16(0.5)) * w

    ids = b_ref[0, 0, :]
    base = meta_ref[0, i]
    span = meta_ref[1, i]

    @pl.when(i == 0)
    def _init():
        acc_ref[...] = jnp.zeros_like(acc_ref)

    @pl.when(span < LOCAL)
    def _local():
        onehot = (jax.lax.broadcasted_iota(jnp.int32, (LOCAL, BLK), 0)
                  == (ids - base)[None, :]).astype(jnp.bfloat16)
        part = jax.lax.dot(onehot, h, preferred_element_type=jnp.float32)
        acc_ref[pl.ds(base, LOCAL), :] += part

    @pl.when(span >= LOCAL)
    def _full():
        onehot = (jax.lax.broadcasted_iota(jnp.int32, (G, BLK), 0)
                  == ids[None, :]).astype(jnp.bfloat16)
        part = jax.lax.dot(onehot, h, preferred_element_type=jnp.float32)
        acc_ref[pl.ds(0, G), :] += part

    @pl.when(i == pl.num_programs(0) - 1)
    def _final():
        acc = acc_ref[pl.ds(0, G), :]
        m = jnp.maximum(jax.lax.dot(acc, Wm1[...], preferred_element_type=jnp.float32)
                        + bm1[...], 0.0)
        out_ref[...] = (jax.lax.dot(m, Wm2[...], preferred_element_type=jnp.float32)
                        + bm2[...])


@functools.partial(jax.jit, static_argnums=(2,))
def _run(x, batch_i32, nblk, Ww1, bw1, Ww2, bw2, Wg1, bg1, Wg2, bg2,
         Wm1, bm1, Wm2, bm2):
    b3 = batch_i32.reshape(nblk, 1, BLK)
    W1 = jnp.concatenate([Ww1, Wg1], axis=1).astype(jnp.bfloat16)
    b1 = jnp.concatenate([bw1, bg1]).reshape(1, 2 * D).astype(jnp.bfloat16)
    zero = jnp.zeros((D, D), jnp.float32)
    W2 = jnp.block([[Ww2, zero], [zero, Wg2]]).astype(jnp.bfloat16)
    b2 = jnp.concatenate([bw2, bg2]).reshape(1, 2 * D).astype(jnp.bfloat16)

    first = batch_i32[::BLK]
    last = batch_i32[BLK - 1::BLK]
    base = jnp.bitwise_and(first, -8)
    meta = jnp.stack([base, last - base])

    row_spec = pl.BlockSpec((BLK, D), lambda i, m: (i, 0))
    id_spec = pl.BlockSpec((1, 1, BLK), lambda i, m: (i, 0, 0))
    full = lambda *shape: pl.BlockSpec(shape, lambda i, m: (0,) * len(shape))
    grid_spec = pltpu.PrefetchScalarGridSpec(
        num_scalar_prefetch=1,
        grid=(nblk,),
        in_specs=[row_spec, id_spec,
                  full(D, 2 * D), full(1, 2 * D), full(2 * D, 2 * D),
                  full(1, 2 * D), full(D, D), full(1, D), full(D, D),
                  full(1, D)],
        out_specs=full(G, D),
        scratch_shapes=[pltpu.VMEM((G + LOCAL, D), jnp.float32)],
    )
    return pl.pallas_call(
        _fused_body,
        grid_spec=grid_spec,
        out_shape=jax.ShapeDtypeStruct((G, D), jnp.float32),
    )(meta, x, b3, W1, b1, W2, b2,
      Wm1, bm1.reshape(1, D), Wm2, bm2.reshape(1, D))


def kernel(x, batch, dim, Ww1, bw1, Ww2, bw2, Wg1, bg1, Wg2, bg2,
           Wm1, bm1, Wm2, bm2):
    del dim
    batch_i32 = batch.astype(jnp.int32)
    assert x.shape == (N, D) and N % BLK == 0
    return _run(x, batch_i32, N // BLK, Ww1, bw1, Ww2, bw2,
                Wg1, bg1, Wg2, bg2, Wm1, bm1, Wm2, bm2)
